# baseline (speedup 1.0000x reference)
import functools

import jax
import jax.numpy as jnp
from jax import lax
from jax.experimental import pallas as pl
from jax.experimental.pallas import tpu as pltpu
from jax.experimental.pallas import tpu_sc as plsc

B = 16384
D = 16
NC = 2
NS = 16
NW = NC * NS
B_PER_W = B // NW


def _build():
    mesh = plsc.VectorSubcoreMesh(core_axis_name="c", subcore_axis_name="s")

    @functools.partial(
        pl.kernel,
        mesh=mesh,
        out_type=jax.ShapeDtypeStruct((B, 2, D), jnp.float32),
        compiler_params=pltpu.CompilerParams(use_tc_tiling_on_sc=False),
        scratch_types=[
            pltpu.VMEM((B_PER_W,), jnp.int32),
            pltpu.VMEM((B_PER_W,), jnp.int32),
            pltpu.VMEM((B_PER_W, D), jnp.float32),
            pltpu.VMEM((B_PER_W, D), jnp.float32),
            pltpu.SemaphoreType.DMA,
            pltpu.SemaphoreType.DMA,
        ],
    )
    def emb_lookup(x0_hbm, x1_hbm, uid_hbm, iid_hbm, out_hbm,
                   idx_u, idx_i, rows_u, rows_i, sem_u, sem_i):
        wid = lax.axis_index("s") * NC + lax.axis_index("c")
        base = wid * B_PER_W
        pltpu.sync_copy(x0_hbm.at[pl.ds(base, B_PER_W)], idx_u)
        pltpu.sync_copy(x1_hbm.at[pl.ds(base, B_PER_W)], idx_i)
        cu = pltpu.async_copy(uid_hbm.at[idx_u], rows_u, sem_u)
        ci = pltpu.async_copy(iid_hbm.at[idx_i], rows_i, sem_i)
        cu.wait()
        ci.wait()
        pltpu.sync_copy(rows_u, out_hbm.at[pl.ds(base, B_PER_W), 0])
        pltpu.sync_copy(rows_i, out_hbm.at[pl.ds(base, B_PER_W), 1])

    return emb_lookup


_emb_lookup = _build()


@jax.jit
def kernel(x, uid_table, iid_table):
    x0 = x[:, 0]
    x1 = x[:, 1]
    return _emb_lookup(x0, x1, uid_table, iid_table)
